# baseline (speedup 1.0000x reference)
import jax
import jax.numpy as jnp
from jax.experimental import pallas as pl
from jax.experimental.pallas import tpu as pltpu

D_MODEL = 768
NUM_EXPERTS = 64
INV_TEMPERATURE = 10.0
BLOCK_T = 8192


def _router_body(x_ref, w_ref, bc_ref, logits_ref, probs_ref, idx_ref):
    w = w_ref[...]
    xs = x_ref[...]
    acc_t = jax.lax.dot_general(
        w, xs, (((1,), (1,)), ((), ())),
        preferred_element_type=jnp.float32,
    )
    lgt = (acc_t + bc_ref[...]) * INV_TEMPERATURE
    logits_ref[...] = lgt

    iota = jax.lax.broadcasted_iota(jnp.int32, lgt.shape, 0)
    big = jnp.int32(NUM_EXPERTS)
    neg_inf = jnp.float32(-jnp.inf)

    m1 = jnp.max(lgt, axis=0, keepdims=True)
    i1 = jnp.min(jnp.where(lgt == m1, iota, big), axis=0, keepdims=True)
    masked = jnp.where(iota == i1, neg_inf, lgt)
    m2 = jnp.max(masked, axis=0, keepdims=True)
    i2 = jnp.min(jnp.where(masked == m2, iota, big), axis=0, keepdims=True)

    p1 = 1.0 / (1.0 + jnp.exp(m2 - m1))
    probs_ref[0:1, :] = p1
    probs_ref[1:2, :] = 1.0 - p1
    idx_ref[0:1, :] = i1
    idx_ref[1:2, :] = i2


@jax.jit
def kernel(x, W, b):
    n_tokens = x.shape[0]
    grid = (n_tokens // BLOCK_T,)
    out_shapes = (
        jax.ShapeDtypeStruct((NUM_EXPERTS, n_tokens), jnp.float32),
        jax.ShapeDtypeStruct((2, n_tokens), jnp.float32),
        jax.ShapeDtypeStruct((2, n_tokens), jnp.int32),
    )
    logits_t, probs_t, idx_t = pl.pallas_call(
        _router_body,
        grid=grid,
        in_specs=[
            pl.BlockSpec((BLOCK_T, D_MODEL), lambda i: (i, 0)),
            pl.BlockSpec((NUM_EXPERTS, D_MODEL), lambda i: (0, 0)),
            pl.BlockSpec((NUM_EXPERTS, 1), lambda i: (0, 0)),
        ],
        out_specs=(
            pl.BlockSpec((NUM_EXPERTS, BLOCK_T), lambda i: (0, i)),
            pl.BlockSpec((2, BLOCK_T), lambda i: (0, i)),
            pl.BlockSpec((2, BLOCK_T), lambda i: (0, i)),
        ),
        out_shape=out_shapes,
        compiler_params=pltpu.CompilerParams(
            dimension_semantics=("parallel",)
        ),
    )(x, W, b.reshape(NUM_EXPERTS, 1))
    return logits_t.T, probs_t.T, idx_t.T


# speedup vs baseline: 1.0214x; 1.0214x over previous
import jax
import jax.numpy as jnp
from jax.experimental import pallas as pl
from jax.experimental.pallas import tpu as pltpu

D_MODEL = 768
NUM_EXPERTS = 64
INV_TEMPERATURE = 10.0
BLOCK_T = 2048


def _router_body(x_ref, w_ref, bc_ref, logits_ref, probs_ref, idx_ref):
    w = w_ref[...]
    xs = x_ref[...]
    acc_t = jax.lax.dot_general(
        w, xs, (((1,), (1,)), ((), ())),
        preferred_element_type=jnp.float32,
    )
    lgt = (acc_t + bc_ref[...]) * INV_TEMPERATURE
    logits_ref[...] = lgt

    iota = jax.lax.broadcasted_iota(jnp.int32, lgt.shape, 0)
    big = jnp.int32(NUM_EXPERTS)
    neg_inf = jnp.float32(-jnp.inf)

    m1 = jnp.max(lgt, axis=0, keepdims=True)
    i1 = jnp.min(jnp.where(lgt == m1, iota, big), axis=0, keepdims=True)
    masked = jnp.where(iota == i1, neg_inf, lgt)
    m2 = jnp.max(masked, axis=0, keepdims=True)
    i2 = jnp.min(jnp.where(masked == m2, iota, big), axis=0, keepdims=True)

    p1 = 1.0 / (1.0 + jnp.exp(m2 - m1))
    probs_ref[0:1, :] = p1
    probs_ref[1:2, :] = 1.0 - p1
    idx_ref[0:1, :] = i1
    idx_ref[1:2, :] = i2


@jax.jit
def kernel(x, W, b):
    n_tokens = x.shape[0]
    grid = (n_tokens // BLOCK_T,)
    out_shapes = (
        jax.ShapeDtypeStruct((NUM_EXPERTS, n_tokens), jnp.float32),
        jax.ShapeDtypeStruct((2, n_tokens), jnp.float32),
        jax.ShapeDtypeStruct((2, n_tokens), jnp.int32),
    )
    logits_t, probs_t, idx_t = pl.pallas_call(
        _router_body,
        grid=grid,
        in_specs=[
            pl.BlockSpec((BLOCK_T, D_MODEL), lambda i: (i, 0)),
            pl.BlockSpec((NUM_EXPERTS, D_MODEL), lambda i: (0, 0)),
            pl.BlockSpec((NUM_EXPERTS, 1), lambda i: (0, 0)),
        ],
        out_specs=(
            pl.BlockSpec((NUM_EXPERTS, BLOCK_T), lambda i: (0, i)),
            pl.BlockSpec((2, BLOCK_T), lambda i: (0, i)),
            pl.BlockSpec((2, BLOCK_T), lambda i: (0, i)),
        ),
        out_shape=out_shapes,
        compiler_params=pltpu.CompilerParams(
            dimension_semantics=("parallel",)
        ),
    )(x, W, b.reshape(NUM_EXPERTS, 1))
    return logits_t.T, probs_t.T, idx_t.T


# speedup vs baseline: 1.0412x; 1.0194x over previous
import jax
import jax.numpy as jnp
from jax.experimental import pallas as pl
from jax.experimental.pallas import tpu as pltpu

D_MODEL = 768
HALF_D = D_MODEL // 2
NUM_EXPERTS = 64
INV_TEMPERATURE = 10.0
BLOCK_T = 4096


def _router_body(xlo_ref, xhi_ref, w_ref, bc_ref, logits_ref, probs_ref, idx_ref):
    w = w_ref[...]
    acc_t = jax.lax.dot_general(
        w[:, :HALF_D], xlo_ref[...], (((1,), (1,)), ((), ())),
        preferred_element_type=jnp.float32,
    ) + jax.lax.dot_general(
        w[:, HALF_D:], xhi_ref[...], (((1,), (1,)), ((), ())),
        preferred_element_type=jnp.float32,
    )
    lgt = (acc_t + bc_ref[...]) * INV_TEMPERATURE
    logits_ref[...] = lgt

    iota = jax.lax.broadcasted_iota(jnp.int32, lgt.shape, 0)
    big = jnp.int32(NUM_EXPERTS)
    neg_inf = jnp.float32(-jnp.inf)

    m1 = jnp.max(lgt, axis=0, keepdims=True)
    i1 = jnp.min(jnp.where(lgt == m1, iota, big), axis=0, keepdims=True)
    masked = jnp.where(iota == i1, neg_inf, lgt)
    m2 = jnp.max(masked, axis=0, keepdims=True)
    i2 = jnp.min(jnp.where(masked == m2, iota, big), axis=0, keepdims=True)

    p1 = 1.0 / (1.0 + jnp.exp(m2 - m1))
    probs_ref[0:1, :] = p1
    probs_ref[1:2, :] = 1.0 - p1
    idx_ref[0:1, :] = i1
    idx_ref[1:2, :] = i2


@jax.jit
def kernel(x, W, b):
    n_tokens = x.shape[0]
    grid = (n_tokens // BLOCK_T,)
    out_shapes = (
        jax.ShapeDtypeStruct((NUM_EXPERTS, n_tokens), jnp.float32),
        jax.ShapeDtypeStruct((2, n_tokens), jnp.float32),
        jax.ShapeDtypeStruct((2, n_tokens), jnp.int32),
    )
    logits_t, probs_t, idx_t = pl.pallas_call(
        _router_body,
        grid=grid,
        in_specs=[
            pl.BlockSpec((BLOCK_T, HALF_D), lambda i: (i, 0)),
            pl.BlockSpec((BLOCK_T, HALF_D), lambda i: (i, 1)),
            pl.BlockSpec((NUM_EXPERTS, D_MODEL), lambda i: (0, 0)),
            pl.BlockSpec((NUM_EXPERTS, 1), lambda i: (0, 0)),
        ],
        out_specs=(
            pl.BlockSpec((NUM_EXPERTS, BLOCK_T), lambda i: (0, i)),
            pl.BlockSpec((2, BLOCK_T), lambda i: (0, i)),
            pl.BlockSpec((2, BLOCK_T), lambda i: (0, i)),
        ),
        out_shape=out_shapes,
        compiler_params=pltpu.CompilerParams(
            dimension_semantics=("parallel",)
        ),
    )(x, x, W, b.reshape(NUM_EXPERTS, 1))
    return logits_t.T, probs_t.T, idx_t.T


# speedup vs baseline: 1.0491x; 1.0076x over previous
import jax
import jax.numpy as jnp
from jax.experimental import pallas as pl
from jax.experimental.pallas import tpu as pltpu

D_MODEL = 768
NUM_EXPERTS = 64
INV_TEMPERATURE = 10.0
BLOCK_T = 4096


def _router_body(x_ref, w_ref, bc_ref, logits_ref, probs_ref, idx_ref):
    w = w_ref[...]
    xs = x_ref[...]
    acc_t = jax.lax.dot_general(
        w, xs, (((1,), (1,)), ((), ())),
        preferred_element_type=jnp.float32,
    )
    lgt = (acc_t + bc_ref[...]) * INV_TEMPERATURE
    logits_ref[...] = lgt

    iota = jax.lax.broadcasted_iota(jnp.int32, lgt.shape, 0)
    big = jnp.int32(NUM_EXPERTS)
    neg_inf = jnp.float32(-jnp.inf)

    m1 = jnp.max(lgt, axis=0, keepdims=True)
    i1 = jnp.min(jnp.where(lgt == m1, iota, big), axis=0, keepdims=True)
    masked = jnp.where(iota == i1, neg_inf, lgt)
    m2 = jnp.max(masked, axis=0, keepdims=True)
    i2 = jnp.min(jnp.where(masked == m2, iota, big), axis=0, keepdims=True)

    p1 = 1.0 / (1.0 + jnp.exp(m2 - m1))
    probs_ref[0:1, :] = p1
    probs_ref[1:2, :] = 1.0 - p1
    idx_ref[0:1, :] = i1
    idx_ref[1:2, :] = i2


@jax.jit
def kernel(x, W, b):
    n_tokens = x.shape[0]
    grid = (n_tokens // BLOCK_T,)
    out_shapes = (
        jax.ShapeDtypeStruct((NUM_EXPERTS, n_tokens), jnp.float32),
        jax.ShapeDtypeStruct((2, n_tokens), jnp.float32),
        jax.ShapeDtypeStruct((2, n_tokens), jnp.int32),
    )
    logits_t, probs_t, idx_t = pl.pallas_call(
        _router_body,
        grid=grid,
        in_specs=[
            pl.BlockSpec((BLOCK_T, D_MODEL), lambda i: (i, 0)),
            pl.BlockSpec((NUM_EXPERTS, D_MODEL), lambda i: (0, 0)),
            pl.BlockSpec((NUM_EXPERTS, 1), lambda i: (0, 0)),
        ],
        out_specs=(
            pl.BlockSpec((NUM_EXPERTS, BLOCK_T), lambda i: (0, i)),
            pl.BlockSpec((2, BLOCK_T), lambda i: (0, i)),
            pl.BlockSpec((2, BLOCK_T), lambda i: (0, i)),
        ),
        out_shape=out_shapes,
        compiler_params=pltpu.CompilerParams(
            dimension_semantics=("parallel",)
        ),
    )(x, W, b.reshape(NUM_EXPERTS, 1))
    return logits_t.T, probs_t.T, idx_t.T
